# baseline (speedup 1.0000x reference)
import jax
import jax.numpy as jnp
from jax.experimental import pallas as pl
from jax.experimental.pallas import tpu as pltpu


def _mask_body(m_ref, x_ref, o_ref):
    i = pl.program_id(0)
    o_ref[...] = jnp.where(m_ref[i] != 0, x_ref[...], jnp.zeros_like(x_ref))


def kernel(x, mask):
    B, K, C, H, W, Z = x.shape
    S = C * H * W * Z
    ROWS = 2048
    COLS = S // ROWS
    xf = x.reshape(B * K, ROWS, COLS)
    m_i32 = mask.reshape(B * K).astype(jnp.int32)

    BR = 512
    grid = (B * K, ROWS // BR)

    out = pl.pallas_call(
        _mask_body,
        grid_spec=pltpu.PrefetchScalarGridSpec(
            num_scalar_prefetch=1,
            grid=grid,
            in_specs=[pl.BlockSpec((1, BR, COLS), lambda i, j, m: (i, j, 0))],
            out_specs=pl.BlockSpec((1, BR, COLS), lambda i, j, m: (i, j, 0)),
        ),
        out_shape=jax.ShapeDtypeStruct((B * K, ROWS, COLS), x.dtype),
    )(m_i32, xf)
    return out.reshape(B, K * C, H, W, Z)
